# Initial kernel scaffold; baseline (speedup 1.0000x reference)
import jax
import jax.numpy as jnp
from jax.experimental import pallas as pl


def kernel(queries, keys):
    raise NotImplementedError("write your pallas kernel here")


# speedup vs baseline: 1.2573x; 1.2573x over previous
import jax
import jax.numpy as jnp
from jax.experimental import pallas as pl
from jax.experimental.pallas import tpu as pltpu


def _body(q_ref, k_ref, d_ref, i_ref):
    step = pl.program_id(0)
    n_keys_total = pl.num_programs(0) * k_ref.shape[0]

    q = q_ref[:, :]
    k = k_ref[:, :]

    dots = jax.lax.dot_general(
        k, q, (((1,), (1,)), ((), ())),
        preferred_element_type=jnp.float32)
    ksq = jnp.sum(k * k, axis=1, keepdims=True)
    dist = ksq - 2.0 * dots

    lmin = jnp.min(dist, axis=0, keepdims=True)
    rows = jax.lax.broadcasted_iota(jnp.int32, dist.shape, 0)
    lidx = jnp.min(jnp.where(dist == lmin, rows, n_keys_total),
                   axis=0, keepdims=True)
    lmin = lmin.T
    lidx = lidx.T + step * k_ref.shape[0]

    @pl.when(step == 0)
    def _init():
        d_ref[:, :] = jnp.full(d_ref.shape, jnp.inf, jnp.float32)
        i_ref[:, :] = jnp.zeros(i_ref.shape, jnp.int32)

    better = lmin < d_ref[:, :]
    i_ref[:, :] = jnp.where(better, lidx, i_ref[:, :])
    d_ref[:, :] = jnp.where(better, lmin, d_ref[:, :])


def kernel(queries, keys):
    q_n, dim = queries.shape
    n_keys = keys.shape[0]
    bk = 8000
    grid = (n_keys // bk,)

    d_out, i_out = pl.pallas_call(
        _body,
        grid=grid,
        in_specs=[
            pl.BlockSpec((q_n, dim), lambda i: (0, 0)),
            pl.BlockSpec((bk, dim), lambda i: (i, 0)),
        ],
        out_specs=[
            pl.BlockSpec((q_n, 1), lambda i: (0, 0)),
            pl.BlockSpec((q_n, 1), lambda i: (0, 0)),
        ],
        out_shape=[
            jax.ShapeDtypeStruct((q_n, 1), jnp.float32),
            jax.ShapeDtypeStruct((q_n, 1), jnp.int32),
        ],
        compiler_params=pltpu.CompilerParams(
            dimension_semantics=("arbitrary",)),
    )(queries, keys)

    qsq = jnp.sum(queries * queries, axis=1, keepdims=True)
    return (d_out + qsq, i_out)


# speedup vs baseline: 1.5871x; 1.2623x over previous
import jax
import jax.numpy as jnp
from jax.experimental import pallas as pl
from jax.experimental.pallas import tpu as pltpu


def _body(rhs_ref, k_ref, d_ref, i_ref):
    step = pl.program_id(0)
    bk = k_ref.shape[0]
    n_keys_total = pl.num_programs(0) * bk

    k = k_ref[:, :]
    lhs = jnp.concatenate([k, k * k], axis=1)
    dist = jax.lax.dot_general(
        lhs, rhs_ref[:, :], (((1,), (0,)), ((), ())),
        preferred_element_type=jnp.float32)
    dist_t = dist.T

    lmin = jnp.min(dist_t, axis=1, keepdims=True)
    cols = jax.lax.broadcasted_iota(jnp.int32, dist_t.shape, 1)
    lidx = jnp.min(jnp.where(dist_t == lmin, cols, n_keys_total),
                   axis=1, keepdims=True) + step * bk

    @pl.when(step == 0)
    def _init():
        d_ref[:, :] = jnp.full(d_ref.shape, jnp.inf, jnp.float32)
        i_ref[:, :] = jnp.zeros(i_ref.shape, jnp.int32)

    better = lmin < d_ref[:, :]
    i_ref[:, :] = jnp.where(better, lidx, i_ref[:, :])
    d_ref[:, :] = jnp.where(better, lmin, d_ref[:, :])


def kernel(queries, keys):
    q_n, dim = queries.shape
    n_keys = keys.shape[0]
    bk = 8000
    grid = (n_keys // bk,)

    rhs = jnp.concatenate(
        [-2.0 * queries.T, jnp.ones((dim, q_n), jnp.float32)], axis=0)

    d_out, i_out = pl.pallas_call(
        _body,
        grid=grid,
        in_specs=[
            pl.BlockSpec((2 * dim, q_n), lambda i: (0, 0)),
            pl.BlockSpec((bk, dim), lambda i: (i, 0)),
        ],
        out_specs=[
            pl.BlockSpec((q_n, 1), lambda i: (0, 0)),
            pl.BlockSpec((q_n, 1), lambda i: (0, 0)),
        ],
        out_shape=[
            jax.ShapeDtypeStruct((q_n, 1), jnp.float32),
            jax.ShapeDtypeStruct((q_n, 1), jnp.int32),
        ],
        compiler_params=pltpu.CompilerParams(
            dimension_semantics=("arbitrary",)),
    )(rhs, keys)

    qsq = jnp.sum(queries * queries, axis=1, keepdims=True)
    return (d_out + qsq, i_out)


# speedup vs baseline: 2.0535x; 1.2939x over previous
import jax
import jax.numpy as jnp
from jax.experimental import pallas as pl
from jax.experimental.pallas import tpu as pltpu


def _body(rhs_ref, k_ref, d_ref, i_ref):
    step = pl.program_id(0)
    bk = k_ref.shape[0]
    n_keys_total = pl.num_programs(0) * bk

    k = k_ref[:, :]
    lhs = jnp.concatenate([k, k * k], axis=1)
    dist = jax.lax.dot_general(
        lhs, rhs_ref[:, :], (((1,), (0,)), ((), ())),
        preferred_element_type=jnp.float32)
    dist_t = dist.T

    lmin = jnp.min(dist_t, axis=1, keepdims=True)
    cols = jax.lax.broadcasted_iota(jnp.int32, dist_t.shape, 1)
    lidx = jnp.min(jnp.where(dist_t == lmin, cols, n_keys_total),
                   axis=1, keepdims=True) + step * bk

    @pl.when(step == 0)
    def _init():
        d_ref[:, :] = jnp.full(d_ref.shape, jnp.inf, jnp.float32)
        i_ref[:, :] = jnp.zeros(i_ref.shape, jnp.int32)

    better = lmin < d_ref[:, :]
    i_ref[:, :] = jnp.where(better, lidx, i_ref[:, :])
    d_ref[:, :] = jnp.where(better, lmin, d_ref[:, :])


def kernel(queries, keys):
    q_n, dim = queries.shape
    n_keys = keys.shape[0]
    bk = 20000
    grid = (n_keys // bk,)

    rhs = jnp.concatenate(
        [-2.0 * queries.T, jnp.ones((dim, q_n), jnp.float32)], axis=0)

    d_out, i_out = pl.pallas_call(
        _body,
        grid=grid,
        in_specs=[
            pl.BlockSpec((2 * dim, q_n), lambda i: (0, 0)),
            pl.BlockSpec((bk, dim), lambda i: (i, 0)),
        ],
        out_specs=[
            pl.BlockSpec((q_n, 1), lambda i: (0, 0)),
            pl.BlockSpec((q_n, 1), lambda i: (0, 0)),
        ],
        out_shape=[
            jax.ShapeDtypeStruct((q_n, 1), jnp.float32),
            jax.ShapeDtypeStruct((q_n, 1), jnp.int32),
        ],
        compiler_params=pltpu.CompilerParams(
            dimension_semantics=("arbitrary",)),
    )(rhs, keys)

    qsq = jnp.sum(queries * queries, axis=1, keepdims=True)
    return (d_out + qsq, i_out)
